# baseline (speedup 1.0000x reference)
import functools

import jax
import jax.numpy as jnp
from jax import lax
from jax.experimental import pallas as pl
from jax.experimental.pallas import tpu as pltpu
from jax.experimental.pallas import tpu_sc as plsc

LATENT = 128
EPS = 1e-5

_NC = 2
_NS = 16
_NW = _NC * _NS


def _proj_body(x_ref, wi_ref, wj_ref, a_ref, b_ref):
    x = x_ref[...]
    a_ref[...] = jnp.dot(x, wi_ref[...], preferred_element_type=jnp.float32)
    b_ref[...] = jnp.dot(x, wj_ref[...], preferred_element_type=jnp.float32)


def _project(x, w_dst, w_src):
    n = x.shape[0]
    blk = 2000
    grid = n // blk
    return pl.pallas_call(
        _proj_body,
        grid=(grid,),
        in_specs=[
            pl.BlockSpec((blk, LATENT), lambda i: (i, 0)),
            pl.BlockSpec((LATENT, LATENT), lambda i: (0, 0)),
            pl.BlockSpec((LATENT, LATENT), lambda i: (0, 0)),
        ],
        out_specs=[
            pl.BlockSpec((blk, LATENT), lambda i: (i, 0)),
            pl.BlockSpec((blk, LATENT), lambda i: (i, 0)),
        ],
        out_shape=[
            jax.ShapeDtypeStruct((n, LATENT), jnp.float32),
            jax.ShapeDtypeStruct((n, LATENT), jnp.float32),
        ],
    )(x, w_dst, w_src)


def _edge_body(g_ref, ea_ref, w0_ref, b0_ref, w1_ref, b1_ref,
               w2_ref, b2_ref, gam_ref, bet_ref, out_ref):
    ea = ea_ref[...]
    h = jnp.dot(ea, w0_ref[...], preferred_element_type=jnp.float32)
    h = h + g_ref[...] + b0_ref[...]
    h = jnp.maximum(h, 0.0)
    h = jnp.dot(h, w1_ref[...], preferred_element_type=jnp.float32) + b1_ref[...]
    h = jnp.maximum(h, 0.0)
    h = jnp.dot(h, w2_ref[...], preferred_element_type=jnp.float32) + b2_ref[...]
    mu = jnp.mean(h, axis=-1, keepdims=True)
    var = jnp.mean((h - mu) ** 2, axis=-1, keepdims=True)
    h = (h - mu) * lax.rsqrt(var + EPS) * gam_ref[...] + bet_ref[...]
    out_ref[...] = h + ea


def _edge_mlp(g, ea, p, ea_off_blocks=0):
    e = g.shape[0]
    blk = 2000
    grid = e // blk
    row = lambda v: v.reshape(1, LATENT)
    wspec = pl.BlockSpec((LATENT, LATENT), lambda i: (0, 0))
    vspec = pl.BlockSpec((1, LATENT), lambda i: (0, 0))
    espec = pl.BlockSpec((blk, LATENT), lambda i: (i, 0))
    easpec = pl.BlockSpec((blk, LATENT), lambda i: (i + ea_off_blocks, 0))
    return pl.pallas_call(
        _edge_body,
        grid=(grid,),
        in_specs=[espec, easpec,
                  wspec, vspec, wspec, vspec, wspec, vspec, vspec, vspec],
        out_specs=espec,
        out_shape=jax.ShapeDtypeStruct((e, LATENT), jnp.float32),
    )(g, ea, p['W0'][2 * LATENT:], row(p['b0']), p['W1'], row(p['b1']),
      p['W2'], row(p['b2']), row(p['gamma']), row(p['beta']))


def _node_body(x_ref, p0_ref, p1_ref, p2_ref, p3_ref, v0a_ref, v0b_ref,
               c0_ref, v1_ref, c1_ref, v2_ref, c2_ref, gam_ref, bet_ref,
               wi_ref, wj_ref, out_ref, a_ref, b_ref, *, with_next):
    x = x_ref[...]
    agg = (p0_ref[...] + p1_ref[...]) + (p2_ref[...] + p3_ref[...])
    h = jnp.dot(x, v0a_ref[...], preferred_element_type=jnp.float32)
    h = h + jnp.dot(agg, v0b_ref[...], preferred_element_type=jnp.float32)
    h = h + c0_ref[...]
    h = jnp.maximum(h, 0.0)
    h = jnp.dot(h, v1_ref[...], preferred_element_type=jnp.float32) + c1_ref[...]
    h = jnp.maximum(h, 0.0)
    h = jnp.dot(h, v2_ref[...], preferred_element_type=jnp.float32) + c2_ref[...]
    mu = jnp.mean(h, axis=-1, keepdims=True)
    var = jnp.mean((h - mu) ** 2, axis=-1, keepdims=True)
    h = (h - mu) * lax.rsqrt(var + EPS) * gam_ref[...] + bet_ref[...]
    xn = h + x
    out_ref[...] = xn
    if with_next:
        a_ref[...] = jnp.dot(xn, wi_ref[...], preferred_element_type=jnp.float32)
        b_ref[...] = jnp.dot(xn, wj_ref[...], preferred_element_type=jnp.float32)


def _node_mlp(x, parts, p, wnext):
    n = x.shape[0]
    blk = 2000
    grid = n // blk
    with_next = wnext is not None
    row = lambda v: v.reshape(1, LATENT)
    wspec = pl.BlockSpec((LATENT, LATENT), lambda i: (0, 0))
    vspec = pl.BlockSpec((1, LATENT), lambda i: (0, 0))
    nspec = pl.BlockSpec((blk, LATENT), lambda i: (i, 0))
    if with_next:
        wi = wnext[:LATENT]
        wj = wnext[LATENT:2 * LATENT]
    else:
        wi = wj = jnp.zeros((LATENT, LATENT), jnp.float32)
    nls = jax.ShapeDtypeStruct((n, LATENT), jnp.float32)
    outs = pl.pallas_call(
        functools.partial(_node_body, with_next=with_next),
        grid=(grid,),
        in_specs=[nspec, nspec, nspec, nspec, nspec,
                  wspec, wspec, vspec, wspec, vspec, wspec, vspec,
                  vspec, vspec, wspec, wspec],
        out_specs=[nspec, nspec, nspec],
        out_shape=[nls, nls, nls],
    )(x, parts[0], parts[1], parts[2], parts[3],
      p['W0'][:LATENT], p['W0'][LATENT:], row(p['b0']),
      p['W1'], row(p['b1']), p['W2'], row(p['b2']),
      row(p['gamma']), row(p['beta']), wi, wj)
    return outs


def _gather_fused(a, b, src, dst):
    e = src.shape[0]
    assert e % _NW == 0
    e_per_w = e // _NW
    c = next(cc for cc in range(128, 0, -8) if e_per_w % cc == 0)
    nch = e_per_w // c
    npairs = (nch + 1) // 2

    mesh = plsc.VectorSubcoreMesh(core_axis_name="c", subcore_axis_name="s")

    def _add_rows(ra, rb):
        @pl.loop(0, c)
        def _row(r):
            for l in range(LATENT // 16):
                sl = pl.ds(l * 16, 16)
                ra[r, sl] = ra[r, sl] + rb[r, sl]

    @functools.partial(
        pl.kernel,
        out_type=jax.ShapeDtypeStruct((e, LATENT), jnp.float32),
        mesh=mesh,
        scratch_types=[
            pltpu.VMEM((c,), jnp.int32), pltpu.VMEM((c,), jnp.int32),
            pltpu.VMEM((c,), jnp.int32), pltpu.VMEM((c,), jnp.int32),
            pltpu.VMEM((c, LATENT), jnp.float32),
            pltpu.VMEM((c, LATENT), jnp.float32),
            pltpu.VMEM((c, LATENT), jnp.float32),
            pltpu.VMEM((c, LATENT), jnp.float32),
            pltpu.SemaphoreType.DMA, pltpu.SemaphoreType.DMA,
            pltpu.SemaphoreType.DMA, pltpu.SemaphoreType.DMA,
            pltpu.SemaphoreType.DMA, pltpu.SemaphoreType.DMA,
        ],
    )
    def gk(a_hbm, b_hbm, src_hbm, dst_hbm, g_hbm,
           idx_d0, idx_s0, idx_d1, idx_s1, ra0, rb0, ra1, rb1,
           sa0, sb0, sa1, sb1, sw0, sw1):
        wid = lax.axis_index("s") * _NC + lax.axis_index("c")
        base = wid * e_per_w
        slots = ((idx_d0, idx_s0, ra0, rb0, sa0, sb0, sw0),
                 (idx_d1, idx_s1, ra1, rb1, sa1, sb1, sw1))

        def load_and_issue(k, slot):
            idx_d, idx_s, ra, rb, sa, sb, _ = slot
            off = base + k * c
            pltpu.sync_copy(dst_hbm.at[pl.ds(off, c)], idx_d)
            pltpu.sync_copy(src_hbm.at[pl.ds(off, c)], idx_s)
            pltpu.async_copy(a_hbm.at[idx_d], ra, sa)
            pltpu.async_copy(b_hbm.at[idx_s], rb, sb)

        def wait_gathers(slot):
            idx_d, idx_s, ra, rb, sa, sb, _ = slot
            pltpu.make_async_copy(a_hbm.at[idx_d], ra, sa).wait()
            pltpu.make_async_copy(b_hbm.at[idx_s], rb, sb).wait()

        def wait_writeout(k, slot):
            _, _, ra, _, _, _, sw = slot
            off = base + k * c
            pltpu.make_async_copy(ra, g_hbm.at[pl.ds(off, c)], sw).wait()

        load_and_issue(0, slots[0])
        load_and_issue(1, slots[1])

        @pl.loop(0, npairs)
        def _pair(jp):
            j0 = 2 * jp
            for si in range(2):
                j = j0 + si
                slot = slots[si]
                idx_d, idx_s, ra, rb, sa, sb, sw = slot

                @pl.when(j < nch)
                def _():
                    wait_gathers(slot)
                    _add_rows(ra, rb)
                    off = base + j * c
                    pltpu.async_copy(ra, g_hbm.at[pl.ds(off, c)], sw)

                    @pl.when(j + 2 < nch)
                    def _():
                        wait_writeout(j, slot)
                        load_and_issue(j + 2, slot)

                    @pl.when(j + 2 >= nch)
                    def _():
                        wait_writeout(j, slot)

    return gk(a, b, src, dst)


def _scatter_sum(ue, dst, n):
    e = ue.shape[0]
    assert e % _NW == 0
    e_per_w = e // _NW
    c = next(cc for cc in range(128, 0, -8) if e_per_w % cc == 0)
    nch = e_per_w // c
    stripe = -(-n // (_NS * 8)) * 8
    n_pad = stripe * _NS

    mesh = plsc.VectorSubcoreMesh(core_axis_name="c", subcore_axis_name="s")

    @functools.partial(
        pl.kernel,
        out_type=jax.ShapeDtypeStruct((_NC, n_pad, LATENT), jnp.float32),
        mesh=mesh,
        scratch_types=[
            pltpu.VMEM((c,), jnp.int32), pltpu.VMEM((c,), jnp.int32),
            pltpu.VMEM((c, LATENT), jnp.float32),
            pltpu.VMEM((c, LATENT), jnp.float32),
            pltpu.VMEM_SHARED((n_pad, LATENT), jnp.float32),
            pltpu.SemaphoreType.DMA, pltpu.SemaphoreType.DMA,
            pltpu.SemaphoreType.DMA, pltpu.SemaphoreType.DMA,
        ],
    )
    def sk(ue_hbm, dst_hbm, z_hbm, out_hbm, idx0, idx1, r0, r1, acc,
           si0, sr0, si1, sr1):
        cid = lax.axis_index("c")
        sid = lax.axis_index("s")
        wid = sid * _NC + cid
        base = wid * e_per_w
        s0 = sid * stripe
        zcp = pltpu.async_copy(z_hbm, acc.at[pl.ds(s0, stripe)], si0)
        slots = ((idx0, r0, si0, sr0), (idx1, r1, si1, sr1))
        npairs = (nch + 1) // 2

        def issue(k, slot):
            idx, rows, si, sr = slot
            off = base + k * c
            pltpu.async_copy(dst_hbm.at[pl.ds(off, c)], idx, si)
            pltpu.async_copy(ue_hbm.at[pl.ds(off, c)], rows, sr)

        def wait_loads(k, slot):
            idx, rows, si, sr = slot
            off = base + k * c
            pltpu.make_async_copy(dst_hbm.at[pl.ds(off, c)], idx, si).wait()
            pltpu.make_async_copy(ue_hbm.at[pl.ds(off, c)], rows, sr).wait()

        zcp.wait()
        plsc.subcore_barrier()
        issue(0, slots[0])
        issue(1, slots[1])

        @pl.loop(0, npairs)
        def _pair(jp):
            j0 = 2 * jp
            for si_ in range(2):
                j = j0 + si_
                slot = slots[si_]

                @pl.when(j < nch)
                def _():
                    wait_loads(j, slot)
                    pltpu.sync_copy(slot[1], acc.at[slot[0]], add=True)

                    @pl.when(j + 2 < nch)
                    def _():
                        issue(j + 2, slot)

        plsc.subcore_barrier()
        pltpu.sync_copy(acc.at[pl.ds(s0, stripe)],
                        out_hbm.at[cid, pl.ds(s0, stripe)])

    parts = sk(ue, dst, jnp.zeros((stripe, LATENT), jnp.float32))
    return parts[0], parts[1]


def kernel(x, edge_index, edge_attr, params):
    n = x.shape[0]
    e = edge_attr.shape[0]
    eh = e // 2
    src = edge_index[0].astype(jnp.int32)
    dst = edge_index[1].astype(jnp.int32)
    src_h = (src[:eh], src[eh:])
    dst_h = (dst[:eh], dst[eh:])
    steps = len(params)
    a, b = _project(x, params[0]['edge']['W0'][:LATENT],
                    params[0]['edge']['W0'][LATENT:2 * LATENT])
    ea_parts = (edge_attr, edge_attr)
    ea_offs = (0, eh // 2000)
    for s in range(steps):
        p = params[s]
        ue, parts = [], []
        for k in range(2):
            g = _gather_fused(a, b, src_h[k], dst_h[k])
            ue_k = _edge_mlp(g, ea_parts[k], p['edge'], ea_offs[k])
            ue.append(ue_k)
            pk = _scatter_sum(ue_k, dst_h[k], n)
            parts.extend(pk)
        ea_parts = tuple(ue)
        ea_offs = (0, 0)
        wnext = params[s + 1]['edge']['W0'] if s + 1 < steps else None
        x, a, b = _node_mlp(x, parts, p['node'], wnext)
    return (x, jnp.concatenate(ea_parts, axis=0))


# speedup vs baseline: 1.1752x; 1.1752x over previous
import functools

import jax
import jax.numpy as jnp
from jax import lax
from jax.experimental import pallas as pl
from jax.experimental.pallas import tpu as pltpu
from jax.experimental.pallas import tpu_sc as plsc

LATENT = 128
EPS = 1e-5

_NC = 2
_NS = 16
_NW = _NC * _NS


def _proj_body(x_ref, wi_ref, wj_ref, a_ref, b_ref):
    x = x_ref[...]
    a_ref[...] = jnp.dot(x, wi_ref[...], preferred_element_type=jnp.float32)
    b_ref[...] = jnp.dot(x, wj_ref[...], preferred_element_type=jnp.float32)


def _project(x, w_dst, w_src):
    n = x.shape[0]
    blk = 2000
    grid = n // blk
    return pl.pallas_call(
        _proj_body,
        grid=(grid,),
        in_specs=[
            pl.BlockSpec((blk, LATENT), lambda i: (i, 0)),
            pl.BlockSpec((LATENT, LATENT), lambda i: (0, 0)),
            pl.BlockSpec((LATENT, LATENT), lambda i: (0, 0)),
        ],
        out_specs=[
            pl.BlockSpec((blk, LATENT), lambda i: (i, 0)),
            pl.BlockSpec((blk, LATENT), lambda i: (i, 0)),
        ],
        out_shape=[
            jax.ShapeDtypeStruct((n, LATENT), jnp.float32),
            jax.ShapeDtypeStruct((n, LATENT), jnp.float32),
        ],
    )(x, w_dst, w_src)


def _edge_body(g_ref, ea_ref, w0_ref, b0_ref, w1_ref, b1_ref,
               w2_ref, b2_ref, gam_ref, bet_ref, out_ref):
    ea = ea_ref[...]
    h = jnp.dot(ea, w0_ref[...], preferred_element_type=jnp.float32)
    h = h + g_ref[...] + b0_ref[...]
    h = jnp.maximum(h, 0.0)
    h = jnp.dot(h, w1_ref[...], preferred_element_type=jnp.float32) + b1_ref[...]
    h = jnp.maximum(h, 0.0)
    h = jnp.dot(h, w2_ref[...], preferred_element_type=jnp.float32) + b2_ref[...]
    mu = jnp.mean(h, axis=-1, keepdims=True)
    var = jnp.mean((h - mu) ** 2, axis=-1, keepdims=True)
    h = (h - mu) * lax.rsqrt(var + EPS) * gam_ref[...] + bet_ref[...]
    out_ref[...] = h + ea


def _edge_mlp(g, ea, p, ea_off_blocks=0):
    e = g.shape[0]
    blk = 4000
    grid = e // blk
    row = lambda v: v.reshape(1, LATENT)
    wspec = pl.BlockSpec((LATENT, LATENT), lambda i: (0, 0))
    vspec = pl.BlockSpec((1, LATENT), lambda i: (0, 0))
    espec = pl.BlockSpec((blk, LATENT), lambda i: (i, 0))
    easpec = pl.BlockSpec((blk, LATENT), lambda i: (i + ea_off_blocks, 0))
    return pl.pallas_call(
        _edge_body,
        grid=(grid,),
        in_specs=[espec, easpec,
                  wspec, vspec, wspec, vspec, wspec, vspec, vspec, vspec],
        out_specs=espec,
        out_shape=jax.ShapeDtypeStruct((e, LATENT), jnp.float32),
    )(g, ea, p['W0'][2 * LATENT:], row(p['b0']), p['W1'], row(p['b1']),
      p['W2'], row(p['b2']), row(p['gamma']), row(p['beta']))


def _node_body(x_ref, p0_ref, p1_ref, v0a_ref, v0b_ref,
               c0_ref, v1_ref, c1_ref, v2_ref, c2_ref, gam_ref, bet_ref,
               wi_ref, wj_ref, out_ref, a_ref, b_ref, *, with_next):
    x = x_ref[...]
    agg = p0_ref[...] + p1_ref[...]
    h = jnp.dot(x, v0a_ref[...], preferred_element_type=jnp.float32)
    h = h + jnp.dot(agg, v0b_ref[...], preferred_element_type=jnp.float32)
    h = h + c0_ref[...]
    h = jnp.maximum(h, 0.0)
    h = jnp.dot(h, v1_ref[...], preferred_element_type=jnp.float32) + c1_ref[...]
    h = jnp.maximum(h, 0.0)
    h = jnp.dot(h, v2_ref[...], preferred_element_type=jnp.float32) + c2_ref[...]
    mu = jnp.mean(h, axis=-1, keepdims=True)
    var = jnp.mean((h - mu) ** 2, axis=-1, keepdims=True)
    h = (h - mu) * lax.rsqrt(var + EPS) * gam_ref[...] + bet_ref[...]
    xn = h + x
    out_ref[...] = xn
    if with_next:
        a_ref[...] = jnp.dot(xn, wi_ref[...], preferred_element_type=jnp.float32)
        b_ref[...] = jnp.dot(xn, wj_ref[...], preferred_element_type=jnp.float32)


def _node_mlp(x, parts, p, wnext):
    n = x.shape[0]
    blk = 2000
    grid = n // blk
    with_next = wnext is not None
    row = lambda v: v.reshape(1, LATENT)
    wspec = pl.BlockSpec((LATENT, LATENT), lambda i: (0, 0))
    vspec = pl.BlockSpec((1, LATENT), lambda i: (0, 0))
    nspec = pl.BlockSpec((blk, LATENT), lambda i: (i, 0))
    if with_next:
        wi = wnext[:LATENT]
        wj = wnext[LATENT:2 * LATENT]
    else:
        wi = wj = jnp.zeros((LATENT, LATENT), jnp.float32)
    nls = jax.ShapeDtypeStruct((n, LATENT), jnp.float32)
    outs = pl.pallas_call(
        functools.partial(_node_body, with_next=with_next),
        grid=(grid,),
        in_specs=[nspec, nspec, nspec,
                  wspec, wspec, vspec, wspec, vspec, wspec, vspec,
                  vspec, vspec, wspec, wspec],
        out_specs=[nspec, nspec, nspec],
        out_shape=[nls, nls, nls],
    )(x, parts[0], parts[1],
      p['W0'][:LATENT], p['W0'][LATENT:], row(p['b0']),
      p['W1'], row(p['b1']), p['W2'], row(p['b2']),
      row(p['gamma']), row(p['beta']), wi, wj)
    return outs


def _gather_fused(a, b, src, dst):
    e = src.shape[0]
    assert e % _NW == 0
    e_per_w = e // _NW
    c = next(cc for cc in range(128, 0, -8) if e_per_w % cc == 0)
    nch = e_per_w // c
    npairs = (nch + 1) // 2

    mesh = plsc.VectorSubcoreMesh(core_axis_name="c", subcore_axis_name="s")

    def _add_rows(ra, rb):
        @pl.loop(0, c)
        def _row(r):
            for l in range(LATENT // 16):
                sl = pl.ds(l * 16, 16)
                ra[r, sl] = ra[r, sl] + rb[r, sl]

    @functools.partial(
        pl.kernel,
        out_type=jax.ShapeDtypeStruct((e, LATENT), jnp.float32),
        mesh=mesh,
        scratch_types=[
            pltpu.VMEM((c,), jnp.int32), pltpu.VMEM((c,), jnp.int32),
            pltpu.VMEM((c,), jnp.int32), pltpu.VMEM((c,), jnp.int32),
            pltpu.VMEM((c, LATENT), jnp.float32),
            pltpu.VMEM((c, LATENT), jnp.float32),
            pltpu.VMEM((c, LATENT), jnp.float32),
            pltpu.VMEM((c, LATENT), jnp.float32),
            pltpu.SemaphoreType.DMA, pltpu.SemaphoreType.DMA,
            pltpu.SemaphoreType.DMA, pltpu.SemaphoreType.DMA,
            pltpu.SemaphoreType.DMA, pltpu.SemaphoreType.DMA,
        ],
    )
    def gk(a_hbm, b_hbm, src_hbm, dst_hbm, g_hbm,
           idx_d0, idx_s0, idx_d1, idx_s1, ra0, rb0, ra1, rb1,
           sa0, sb0, sa1, sb1, sw0, sw1):
        wid = lax.axis_index("s") * _NC + lax.axis_index("c")
        base = wid * e_per_w
        slots = ((idx_d0, idx_s0, ra0, rb0, sa0, sb0, sw0),
                 (idx_d1, idx_s1, ra1, rb1, sa1, sb1, sw1))

        def load_and_issue(k, slot):
            idx_d, idx_s, ra, rb, sa, sb, _ = slot
            off = base + k * c
            pltpu.sync_copy(dst_hbm.at[pl.ds(off, c)], idx_d)
            pltpu.sync_copy(src_hbm.at[pl.ds(off, c)], idx_s)
            pltpu.async_copy(a_hbm.at[idx_d], ra, sa)
            pltpu.async_copy(b_hbm.at[idx_s], rb, sb)

        def wait_gathers(slot):
            idx_d, idx_s, ra, rb, sa, sb, _ = slot
            pltpu.make_async_copy(a_hbm.at[idx_d], ra, sa).wait()
            pltpu.make_async_copy(b_hbm.at[idx_s], rb, sb).wait()

        def wait_writeout(k, slot):
            _, _, ra, _, _, _, sw = slot
            off = base + k * c
            pltpu.make_async_copy(ra, g_hbm.at[pl.ds(off, c)], sw).wait()

        load_and_issue(0, slots[0])
        load_and_issue(1, slots[1])

        @pl.loop(0, npairs)
        def _pair(jp):
            j0 = 2 * jp
            for si in range(2):
                j = j0 + si
                slot = slots[si]
                idx_d, idx_s, ra, rb, sa, sb, sw = slot

                @pl.when(j < nch)
                def _():
                    wait_gathers(slot)
                    _add_rows(ra, rb)
                    off = base + j * c
                    pltpu.async_copy(ra, g_hbm.at[pl.ds(off, c)], sw)

                    @pl.when(j + 2 < nch)
                    def _():
                        wait_writeout(j, slot)
                        load_and_issue(j + 2, slot)

                    @pl.when(j + 2 >= nch)
                    def _():
                        wait_writeout(j, slot)

    return gk(a, b, src, dst)


def _scatter_sum(ue, dst, n):
    e = ue.shape[0]
    assert e % _NW == 0
    e_per_w = e // _NW
    c = next(cc for cc in range(128, 0, -8) if e_per_w % cc == 0)
    nch = e_per_w // c
    stripe = -(-n // (_NS * 8)) * 8
    n_pad = stripe * _NS

    mesh = plsc.VectorSubcoreMesh(core_axis_name="c", subcore_axis_name="s")

    @functools.partial(
        pl.kernel,
        out_type=jax.ShapeDtypeStruct((_NC, n_pad, LATENT), jnp.float32),
        mesh=mesh,
        scratch_types=[
            pltpu.VMEM((c,), jnp.int32), pltpu.VMEM((c,), jnp.int32),
            pltpu.VMEM((c, LATENT), jnp.float32),
            pltpu.VMEM((c, LATENT), jnp.float32),
            pltpu.VMEM_SHARED((n_pad, LATENT), jnp.float32),
            pltpu.SemaphoreType.DMA, pltpu.SemaphoreType.DMA,
            pltpu.SemaphoreType.DMA, pltpu.SemaphoreType.DMA,
        ],
    )
    def sk(ue_hbm, dst_hbm, z_hbm, out_hbm, idx0, idx1, r0, r1, acc,
           si0, sr0, si1, sr1):
        cid = lax.axis_index("c")
        sid = lax.axis_index("s")
        wid = sid * _NC + cid
        base = wid * e_per_w
        s0 = sid * stripe
        zcp = pltpu.async_copy(z_hbm, acc.at[pl.ds(s0, stripe)], si0)
        slots = ((idx0, r0, si0, sr0), (idx1, r1, si1, sr1))
        npairs = (nch + 1) // 2

        def issue(k, slot):
            idx, rows, si, sr = slot
            off = base + k * c
            pltpu.async_copy(dst_hbm.at[pl.ds(off, c)], idx, si)
            pltpu.async_copy(ue_hbm.at[pl.ds(off, c)], rows, sr)

        def wait_loads(k, slot):
            idx, rows, si, sr = slot
            off = base + k * c
            pltpu.make_async_copy(dst_hbm.at[pl.ds(off, c)], idx, si).wait()
            pltpu.make_async_copy(ue_hbm.at[pl.ds(off, c)], rows, sr).wait()

        zcp.wait()
        plsc.subcore_barrier()
        issue(0, slots[0])
        issue(1, slots[1])

        @pl.loop(0, npairs)
        def _pair(jp):
            j0 = 2 * jp
            for si_ in range(2):
                j = j0 + si_
                slot = slots[si_]

                @pl.when(j < nch)
                def _():
                    wait_loads(j, slot)
                    pltpu.sync_copy(slot[1], acc.at[slot[0]], add=True)

                    @pl.when(j + 2 < nch)
                    def _():
                        issue(j + 2, slot)

        plsc.subcore_barrier()
        pltpu.sync_copy(acc.at[pl.ds(s0, stripe)],
                        out_hbm.at[cid, pl.ds(s0, stripe)])

    parts = sk(ue, dst, jnp.zeros((stripe, LATENT), jnp.float32))
    return parts[0], parts[1]


def kernel(x, edge_index, edge_attr, params):
    n = x.shape[0]
    src = edge_index[0].astype(jnp.int32)
    dst = edge_index[1].astype(jnp.int32)
    steps = len(params)
    a, b = _project(x, params[0]['edge']['W0'][:LATENT],
                    params[0]['edge']['W0'][LATENT:2 * LATENT])
    for s in range(steps):
        p = params[s]
        g = _gather_fused(a, b, src, dst)
        edge_attr = _edge_mlp(g, edge_attr, p['edge'])
        parts = _scatter_sum(edge_attr, dst, n)
        wnext = params[s + 1]['edge']['W0'] if s + 1 < steps else None
        x, a, b = _node_mlp(x, parts, p['node'], wnext)
    return (x, edge_attr)


# speedup vs baseline: 1.2300x; 1.0467x over previous
import functools

import jax
import jax.numpy as jnp
from jax import lax
from jax.experimental import pallas as pl
from jax.experimental.pallas import tpu as pltpu
from jax.experimental.pallas import tpu_sc as plsc

LATENT = 128
EPS = 1e-5

_NC = 2
_NS = 16
_NW = _NC * _NS


def _proj_body(x_ref, wi_ref, wj_ref, a_ref, b_ref):
    x = x_ref[...]
    a_ref[...] = jnp.dot(x, wi_ref[...], preferred_element_type=jnp.float32)
    b_ref[...] = jnp.dot(x, wj_ref[...], preferred_element_type=jnp.float32)


def _project(x, w_dst, w_src):
    n = x.shape[0]
    blk = 2000
    grid = n // blk
    return pl.pallas_call(
        _proj_body,
        grid=(grid,),
        in_specs=[
            pl.BlockSpec((blk, LATENT), lambda i: (i, 0)),
            pl.BlockSpec((LATENT, LATENT), lambda i: (0, 0)),
            pl.BlockSpec((LATENT, LATENT), lambda i: (0, 0)),
        ],
        out_specs=[
            pl.BlockSpec((blk, LATENT), lambda i: (i, 0)),
            pl.BlockSpec((blk, LATENT), lambda i: (i, 0)),
        ],
        out_shape=[
            jax.ShapeDtypeStruct((n, LATENT), jnp.float32),
            jax.ShapeDtypeStruct((n, LATENT), jnp.float32),
        ],
    )(x, w_dst, w_src)


def _edge_body(g_ref, ea_ref, w0_ref, b0_ref, w1_ref, b1_ref,
               w2_ref, b2_ref, gam_ref, bet_ref, out_ref):
    ea = ea_ref[...]
    h = jnp.dot(ea, w0_ref[...], preferred_element_type=jnp.float32)
    h = h + g_ref[...] + b0_ref[...]
    h = jnp.maximum(h, 0.0)
    h = jnp.dot(h, w1_ref[...], preferred_element_type=jnp.float32) + b1_ref[...]
    h = jnp.maximum(h, 0.0)
    h = jnp.dot(h, w2_ref[...], preferred_element_type=jnp.float32) + b2_ref[...]
    mu = jnp.mean(h, axis=-1, keepdims=True)
    var = jnp.mean((h - mu) ** 2, axis=-1, keepdims=True)
    h = (h - mu) * lax.rsqrt(var + EPS) * gam_ref[...] + bet_ref[...]
    out_ref[...] = h + ea


def _edge_mlp(g, ea, p, ea_off_blocks=0):
    e = g.shape[0]
    blk = 8000
    grid = e // blk
    row = lambda v: v.reshape(1, LATENT)
    wspec = pl.BlockSpec((LATENT, LATENT), lambda i: (0, 0))
    vspec = pl.BlockSpec((1, LATENT), lambda i: (0, 0))
    espec = pl.BlockSpec((blk, LATENT), lambda i: (i, 0))
    easpec = pl.BlockSpec((blk, LATENT), lambda i: (i + ea_off_blocks, 0))
    return pl.pallas_call(
        _edge_body,
        grid=(grid,),
        in_specs=[espec, easpec,
                  wspec, vspec, wspec, vspec, wspec, vspec, vspec, vspec],
        out_specs=espec,
        out_shape=jax.ShapeDtypeStruct((e, LATENT), jnp.float32),
    )(g, ea, p['W0'][2 * LATENT:], row(p['b0']), p['W1'], row(p['b1']),
      p['W2'], row(p['b2']), row(p['gamma']), row(p['beta']))


def _node_body(x_ref, p0_ref, p1_ref, v0a_ref, v0b_ref,
               c0_ref, v1_ref, c1_ref, v2_ref, c2_ref, gam_ref, bet_ref,
               wi_ref, wj_ref, out_ref, a_ref, b_ref, *, with_next):
    x = x_ref[...]
    agg = p0_ref[...] + p1_ref[...]
    h = jnp.dot(x, v0a_ref[...], preferred_element_type=jnp.float32)
    h = h + jnp.dot(agg, v0b_ref[...], preferred_element_type=jnp.float32)
    h = h + c0_ref[...]
    h = jnp.maximum(h, 0.0)
    h = jnp.dot(h, v1_ref[...], preferred_element_type=jnp.float32) + c1_ref[...]
    h = jnp.maximum(h, 0.0)
    h = jnp.dot(h, v2_ref[...], preferred_element_type=jnp.float32) + c2_ref[...]
    mu = jnp.mean(h, axis=-1, keepdims=True)
    var = jnp.mean((h - mu) ** 2, axis=-1, keepdims=True)
    h = (h - mu) * lax.rsqrt(var + EPS) * gam_ref[...] + bet_ref[...]
    xn = h + x
    out_ref[...] = xn
    if with_next:
        a_ref[...] = jnp.dot(xn, wi_ref[...], preferred_element_type=jnp.float32)
        b_ref[...] = jnp.dot(xn, wj_ref[...], preferred_element_type=jnp.float32)


def _node_mlp(x, parts, p, wnext):
    n = x.shape[0]
    blk = 2000
    grid = n // blk
    with_next = wnext is not None
    row = lambda v: v.reshape(1, LATENT)
    wspec = pl.BlockSpec((LATENT, LATENT), lambda i: (0, 0))
    vspec = pl.BlockSpec((1, LATENT), lambda i: (0, 0))
    nspec = pl.BlockSpec((blk, LATENT), lambda i: (i, 0))
    if with_next:
        wi = wnext[:LATENT]
        wj = wnext[LATENT:2 * LATENT]
    else:
        wi = wj = jnp.zeros((LATENT, LATENT), jnp.float32)
    nls = jax.ShapeDtypeStruct((n, LATENT), jnp.float32)
    outs = pl.pallas_call(
        functools.partial(_node_body, with_next=with_next),
        grid=(grid,),
        in_specs=[nspec, nspec, nspec,
                  wspec, wspec, vspec, wspec, vspec, wspec, vspec,
                  vspec, vspec, wspec, wspec],
        out_specs=[nspec, nspec, nspec],
        out_shape=[nls, nls, nls],
    )(x, parts[0], parts[1],
      p['W0'][:LATENT], p['W0'][LATENT:], row(p['b0']),
      p['W1'], row(p['b1']), p['W2'], row(p['b2']),
      row(p['gamma']), row(p['beta']), wi, wj)
    return outs


def _gather_fused(a, b, src, dst):
    e = src.shape[0]
    assert e % _NW == 0
    e_per_w = e // _NW
    c = next(cc for cc in range(128, 0, -8) if e_per_w % cc == 0)
    nch = e_per_w // c
    npairs = (nch + 1) // 2

    mesh = plsc.VectorSubcoreMesh(core_axis_name="c", subcore_axis_name="s")

    def _add_rows(ra, rb):
        @pl.loop(0, c)
        def _row(r):
            for l in range(LATENT // 16):
                sl = pl.ds(l * 16, 16)
                ra[r, sl] = ra[r, sl] + rb[r, sl]

    @functools.partial(
        pl.kernel,
        out_type=jax.ShapeDtypeStruct((e, LATENT), jnp.float32),
        mesh=mesh,
        scratch_types=[
            pltpu.VMEM((c,), jnp.int32), pltpu.VMEM((c,), jnp.int32),
            pltpu.VMEM((c,), jnp.int32), pltpu.VMEM((c,), jnp.int32),
            pltpu.VMEM((c, LATENT), jnp.float32),
            pltpu.VMEM((c, LATENT), jnp.float32),
            pltpu.VMEM((c, LATENT), jnp.float32),
            pltpu.VMEM((c, LATENT), jnp.float32),
            pltpu.SemaphoreType.DMA, pltpu.SemaphoreType.DMA,
            pltpu.SemaphoreType.DMA, pltpu.SemaphoreType.DMA,
            pltpu.SemaphoreType.DMA, pltpu.SemaphoreType.DMA,
        ],
    )
    def gk(a_hbm, b_hbm, src_hbm, dst_hbm, g_hbm,
           idx_d0, idx_s0, idx_d1, idx_s1, ra0, rb0, ra1, rb1,
           sa0, sb0, sa1, sb1, sw0, sw1):
        wid = lax.axis_index("s") * _NC + lax.axis_index("c")
        base = wid * e_per_w
        slots = ((idx_d0, idx_s0, ra0, rb0, sa0, sb0, sw0),
                 (idx_d1, idx_s1, ra1, rb1, sa1, sb1, sw1))

        def load_and_issue(k, slot):
            idx_d, idx_s, ra, rb, sa, sb, _ = slot
            off = base + k * c
            pltpu.sync_copy(dst_hbm.at[pl.ds(off, c)], idx_d)
            pltpu.sync_copy(src_hbm.at[pl.ds(off, c)], idx_s)
            pltpu.async_copy(a_hbm.at[idx_d], ra, sa)
            pltpu.async_copy(b_hbm.at[idx_s], rb, sb)

        def wait_gathers(slot):
            idx_d, idx_s, ra, rb, sa, sb, _ = slot
            pltpu.make_async_copy(a_hbm.at[idx_d], ra, sa).wait()
            pltpu.make_async_copy(b_hbm.at[idx_s], rb, sb).wait()

        def wait_writeout(k, slot):
            _, _, ra, _, _, _, sw = slot
            off = base + k * c
            pltpu.make_async_copy(ra, g_hbm.at[pl.ds(off, c)], sw).wait()

        load_and_issue(0, slots[0])
        load_and_issue(1, slots[1])

        @pl.loop(0, npairs)
        def _pair(jp):
            j0 = 2 * jp
            for si in range(2):
                j = j0 + si
                slot = slots[si]
                idx_d, idx_s, ra, rb, sa, sb, sw = slot

                @pl.when(j < nch)
                def _():
                    wait_gathers(slot)
                    _add_rows(ra, rb)
                    off = base + j * c
                    pltpu.async_copy(ra, g_hbm.at[pl.ds(off, c)], sw)

                    @pl.when(j + 2 < nch)
                    def _():
                        wait_writeout(j, slot)
                        load_and_issue(j + 2, slot)

                    @pl.when(j + 2 >= nch)
                    def _():
                        wait_writeout(j, slot)

    return gk(a, b, src, dst)


def _scatter_sum(ue, dst, n):
    e = ue.shape[0]
    assert e % _NW == 0
    e_per_w = e // _NW
    c = next(cc for cc in range(128, 0, -8) if e_per_w % cc == 0)
    nch = e_per_w // c
    stripe = -(-n // (_NS * 8)) * 8
    n_pad = stripe * _NS

    mesh = plsc.VectorSubcoreMesh(core_axis_name="c", subcore_axis_name="s")

    @functools.partial(
        pl.kernel,
        out_type=jax.ShapeDtypeStruct((_NC, n_pad, LATENT), jnp.float32),
        mesh=mesh,
        scratch_types=[
            pltpu.VMEM((c,), jnp.int32), pltpu.VMEM((c,), jnp.int32),
            pltpu.VMEM((c, LATENT), jnp.float32),
            pltpu.VMEM((c, LATENT), jnp.float32),
            pltpu.VMEM_SHARED((n_pad, LATENT), jnp.float32),
            pltpu.SemaphoreType.DMA, pltpu.SemaphoreType.DMA,
            pltpu.SemaphoreType.DMA, pltpu.SemaphoreType.DMA,
        ],
    )
    def sk(ue_hbm, dst_hbm, z_hbm, out_hbm, idx0, idx1, r0, r1, acc,
           si0, sr0, si1, sr1):
        cid = lax.axis_index("c")
        sid = lax.axis_index("s")
        wid = sid * _NC + cid
        base = wid * e_per_w
        s0 = sid * stripe
        zcp = pltpu.async_copy(z_hbm, acc.at[pl.ds(s0, stripe)], si0)
        slots = ((idx0, r0, si0, sr0), (idx1, r1, si1, sr1))
        npairs = (nch + 1) // 2

        def issue(k, slot):
            idx, rows, si, sr = slot
            off = base + k * c
            pltpu.async_copy(dst_hbm.at[pl.ds(off, c)], idx, si)
            pltpu.async_copy(ue_hbm.at[pl.ds(off, c)], rows, sr)

        def wait_loads(k, slot):
            idx, rows, si, sr = slot
            off = base + k * c
            pltpu.make_async_copy(dst_hbm.at[pl.ds(off, c)], idx, si).wait()
            pltpu.make_async_copy(ue_hbm.at[pl.ds(off, c)], rows, sr).wait()

        zcp.wait()
        plsc.subcore_barrier()
        issue(0, slots[0])
        issue(1, slots[1])

        @pl.loop(0, npairs)
        def _pair(jp):
            j0 = 2 * jp
            for si_ in range(2):
                j = j0 + si_
                slot = slots[si_]

                @pl.when(j < nch)
                def _():
                    wait_loads(j, slot)
                    pltpu.sync_copy(slot[1], acc.at[slot[0]], add=True)

                    @pl.when(j + 2 < nch)
                    def _():
                        issue(j + 2, slot)

        plsc.subcore_barrier()
        pltpu.sync_copy(acc.at[pl.ds(s0, stripe)],
                        out_hbm.at[cid, pl.ds(s0, stripe)])

    parts = sk(ue, dst, jnp.zeros((stripe, LATENT), jnp.float32))
    return parts[0], parts[1]


def kernel(x, edge_index, edge_attr, params):
    n = x.shape[0]
    src = edge_index[0].astype(jnp.int32)
    dst = edge_index[1].astype(jnp.int32)
    steps = len(params)
    a, b = _project(x, params[0]['edge']['W0'][:LATENT],
                    params[0]['edge']['W0'][LATENT:2 * LATENT])
    for s in range(steps):
        p = params[s]
        g = _gather_fused(a, b, src, dst)
        edge_attr = _edge_mlp(g, edge_attr, p['edge'])
        parts = _scatter_sum(edge_attr, dst, n)
        wnext = params[s + 1]['edge']['W0'] if s + 1 < steps else None
        x, a, b = _node_mlp(x, parts, p['node'], wnext)
    return (x, edge_attr)


# speedup vs baseline: 1.2411x; 1.0091x over previous
import functools

import jax
import jax.numpy as jnp
from jax import lax
from jax.experimental import pallas as pl
from jax.experimental.pallas import tpu as pltpu
from jax.experimental.pallas import tpu_sc as plsc

LATENT = 128
EPS = 1e-5

_NC = 2
_NS = 16
_NW = _NC * _NS


def _proj_body(x_ref, wi_ref, wj_ref, a_ref, b_ref):
    x = x_ref[...]
    a_ref[...] = jnp.dot(x, wi_ref[...], preferred_element_type=jnp.float32)
    b_ref[...] = jnp.dot(x, wj_ref[...], preferred_element_type=jnp.float32)


def _project(x, w_dst, w_src):
    n = x.shape[0]
    blk = 2000
    grid = n // blk
    return pl.pallas_call(
        _proj_body,
        grid=(grid,),
        in_specs=[
            pl.BlockSpec((blk, LATENT), lambda i: (i, 0)),
            pl.BlockSpec((LATENT, LATENT), lambda i: (0, 0)),
            pl.BlockSpec((LATENT, LATENT), lambda i: (0, 0)),
        ],
        out_specs=[
            pl.BlockSpec((blk, LATENT), lambda i: (i, 0)),
            pl.BlockSpec((blk, LATENT), lambda i: (i, 0)),
        ],
        out_shape=[
            jax.ShapeDtypeStruct((n, LATENT), jnp.float32),
            jax.ShapeDtypeStruct((n, LATENT), jnp.float32),
        ],
    )(x, w_dst, w_src)


def _edge_body(g_ref, ea_ref, w0_ref, b0_ref, w1_ref, b1_ref,
               w2_ref, b2_ref, gam_ref, bet_ref, out_ref):
    ea = ea_ref[...]
    h = jnp.dot(ea, w0_ref[...], preferred_element_type=jnp.float32)
    h = h + g_ref[...] + b0_ref[...]
    h = jnp.maximum(h, 0.0)
    h = jnp.dot(h, w1_ref[...], preferred_element_type=jnp.float32) + b1_ref[...]
    h = jnp.maximum(h, 0.0)
    h = jnp.dot(h, w2_ref[...], preferred_element_type=jnp.float32) + b2_ref[...]
    mu = jnp.mean(h, axis=-1, keepdims=True)
    var = jnp.mean((h - mu) ** 2, axis=-1, keepdims=True)
    h = (h - mu) * lax.rsqrt(var + EPS) * gam_ref[...] + bet_ref[...]
    out_ref[...] = h + ea


def _edge_mlp(g, ea, p, ea_off_blocks=0):
    e = g.shape[0]
    blk = 16000
    grid = e // blk
    row = lambda v: v.reshape(1, LATENT)
    wspec = pl.BlockSpec((LATENT, LATENT), lambda i: (0, 0))
    vspec = pl.BlockSpec((1, LATENT), lambda i: (0, 0))
    espec = pl.BlockSpec((blk, LATENT), lambda i: (i, 0))
    easpec = pl.BlockSpec((blk, LATENT), lambda i: (i + ea_off_blocks, 0))
    return pl.pallas_call(
        _edge_body,
        grid=(grid,),
        in_specs=[espec, easpec,
                  wspec, vspec, wspec, vspec, wspec, vspec, vspec, vspec],
        out_specs=espec,
        out_shape=jax.ShapeDtypeStruct((e, LATENT), jnp.float32),
    )(g, ea, p['W0'][2 * LATENT:], row(p['b0']), p['W1'], row(p['b1']),
      p['W2'], row(p['b2']), row(p['gamma']), row(p['beta']))


def _node_body(x_ref, p0_ref, p1_ref, v0a_ref, v0b_ref,
               c0_ref, v1_ref, c1_ref, v2_ref, c2_ref, gam_ref, bet_ref,
               wi_ref, wj_ref, out_ref, a_ref, b_ref, *, with_next):
    x = x_ref[...]
    agg = p0_ref[...] + p1_ref[...]
    h = jnp.dot(x, v0a_ref[...], preferred_element_type=jnp.float32)
    h = h + jnp.dot(agg, v0b_ref[...], preferred_element_type=jnp.float32)
    h = h + c0_ref[...]
    h = jnp.maximum(h, 0.0)
    h = jnp.dot(h, v1_ref[...], preferred_element_type=jnp.float32) + c1_ref[...]
    h = jnp.maximum(h, 0.0)
    h = jnp.dot(h, v2_ref[...], preferred_element_type=jnp.float32) + c2_ref[...]
    mu = jnp.mean(h, axis=-1, keepdims=True)
    var = jnp.mean((h - mu) ** 2, axis=-1, keepdims=True)
    h = (h - mu) * lax.rsqrt(var + EPS) * gam_ref[...] + bet_ref[...]
    xn = h + x
    out_ref[...] = xn
    if with_next:
        a_ref[...] = jnp.dot(xn, wi_ref[...], preferred_element_type=jnp.float32)
        b_ref[...] = jnp.dot(xn, wj_ref[...], preferred_element_type=jnp.float32)


def _node_mlp(x, parts, p, wnext):
    n = x.shape[0]
    blk = 5000
    grid = n // blk
    with_next = wnext is not None
    row = lambda v: v.reshape(1, LATENT)
    wspec = pl.BlockSpec((LATENT, LATENT), lambda i: (0, 0))
    vspec = pl.BlockSpec((1, LATENT), lambda i: (0, 0))
    nspec = pl.BlockSpec((blk, LATENT), lambda i: (i, 0))
    if with_next:
        wi = wnext[:LATENT]
        wj = wnext[LATENT:2 * LATENT]
    else:
        wi = wj = jnp.zeros((LATENT, LATENT), jnp.float32)
    nls = jax.ShapeDtypeStruct((n, LATENT), jnp.float32)
    outs = pl.pallas_call(
        functools.partial(_node_body, with_next=with_next),
        grid=(grid,),
        in_specs=[nspec, nspec, nspec,
                  wspec, wspec, vspec, wspec, vspec, wspec, vspec,
                  vspec, vspec, wspec, wspec],
        out_specs=[nspec, nspec, nspec],
        out_shape=[nls, nls, nls],
    )(x, parts[0], parts[1],
      p['W0'][:LATENT], p['W0'][LATENT:], row(p['b0']),
      p['W1'], row(p['b1']), p['W2'], row(p['b2']),
      row(p['gamma']), row(p['beta']), wi, wj)
    return outs


def _gather_fused(a, b, src, dst):
    e = src.shape[0]
    assert e % _NW == 0
    e_per_w = e // _NW
    c = next(cc for cc in range(128, 0, -8) if e_per_w % cc == 0)
    nch = e_per_w // c
    npairs = (nch + 1) // 2

    mesh = plsc.VectorSubcoreMesh(core_axis_name="c", subcore_axis_name="s")

    def _add_rows(ra, rb):
        @pl.loop(0, c)
        def _row(r):
            for l in range(LATENT // 16):
                sl = pl.ds(l * 16, 16)
                ra[r, sl] = ra[r, sl] + rb[r, sl]

    @functools.partial(
        pl.kernel,
        out_type=jax.ShapeDtypeStruct((e, LATENT), jnp.float32),
        mesh=mesh,
        scratch_types=[
            pltpu.VMEM((c,), jnp.int32), pltpu.VMEM((c,), jnp.int32),
            pltpu.VMEM((c,), jnp.int32), pltpu.VMEM((c,), jnp.int32),
            pltpu.VMEM((c, LATENT), jnp.float32),
            pltpu.VMEM((c, LATENT), jnp.float32),
            pltpu.VMEM((c, LATENT), jnp.float32),
            pltpu.VMEM((c, LATENT), jnp.float32),
            pltpu.SemaphoreType.DMA, pltpu.SemaphoreType.DMA,
            pltpu.SemaphoreType.DMA, pltpu.SemaphoreType.DMA,
            pltpu.SemaphoreType.DMA, pltpu.SemaphoreType.DMA,
        ],
    )
    def gk(a_hbm, b_hbm, src_hbm, dst_hbm, g_hbm,
           idx_d0, idx_s0, idx_d1, idx_s1, ra0, rb0, ra1, rb1,
           sa0, sb0, sa1, sb1, sw0, sw1):
        wid = lax.axis_index("s") * _NC + lax.axis_index("c")
        base = wid * e_per_w
        slots = ((idx_d0, idx_s0, ra0, rb0, sa0, sb0, sw0),
                 (idx_d1, idx_s1, ra1, rb1, sa1, sb1, sw1))

        def load_and_issue(k, slot):
            idx_d, idx_s, ra, rb, sa, sb, _ = slot
            off = base + k * c
            pltpu.sync_copy(dst_hbm.at[pl.ds(off, c)], idx_d)
            pltpu.sync_copy(src_hbm.at[pl.ds(off, c)], idx_s)
            pltpu.async_copy(a_hbm.at[idx_d], ra, sa)
            pltpu.async_copy(b_hbm.at[idx_s], rb, sb)

        def wait_gathers(slot):
            idx_d, idx_s, ra, rb, sa, sb, _ = slot
            pltpu.make_async_copy(a_hbm.at[idx_d], ra, sa).wait()
            pltpu.make_async_copy(b_hbm.at[idx_s], rb, sb).wait()

        def wait_writeout(k, slot):
            _, _, ra, _, _, _, sw = slot
            off = base + k * c
            pltpu.make_async_copy(ra, g_hbm.at[pl.ds(off, c)], sw).wait()

        load_and_issue(0, slots[0])
        load_and_issue(1, slots[1])

        @pl.loop(0, npairs)
        def _pair(jp):
            j0 = 2 * jp
            for si in range(2):
                j = j0 + si
                slot = slots[si]
                idx_d, idx_s, ra, rb, sa, sb, sw = slot

                @pl.when(j < nch)
                def _():
                    wait_gathers(slot)
                    _add_rows(ra, rb)
                    off = base + j * c
                    pltpu.async_copy(ra, g_hbm.at[pl.ds(off, c)], sw)

                    @pl.when(j + 2 < nch)
                    def _():
                        wait_writeout(j, slot)
                        load_and_issue(j + 2, slot)

                    @pl.when(j + 2 >= nch)
                    def _():
                        wait_writeout(j, slot)

    return gk(a, b, src, dst)


def _scatter_sum(ue, dst, n):
    e = ue.shape[0]
    assert e % _NW == 0
    e_per_w = e // _NW
    c = next(cc for cc in range(128, 0, -8) if e_per_w % cc == 0)
    nch = e_per_w // c
    stripe = -(-n // (_NS * 8)) * 8
    n_pad = stripe * _NS

    mesh = plsc.VectorSubcoreMesh(core_axis_name="c", subcore_axis_name="s")

    @functools.partial(
        pl.kernel,
        out_type=jax.ShapeDtypeStruct((_NC, n_pad, LATENT), jnp.float32),
        mesh=mesh,
        scratch_types=[
            pltpu.VMEM((c,), jnp.int32), pltpu.VMEM((c,), jnp.int32),
            pltpu.VMEM((c, LATENT), jnp.float32),
            pltpu.VMEM((c, LATENT), jnp.float32),
            pltpu.VMEM_SHARED((n_pad, LATENT), jnp.float32),
            pltpu.SemaphoreType.DMA, pltpu.SemaphoreType.DMA,
            pltpu.SemaphoreType.DMA, pltpu.SemaphoreType.DMA,
        ],
    )
    def sk(ue_hbm, dst_hbm, z_hbm, out_hbm, idx0, idx1, r0, r1, acc,
           si0, sr0, si1, sr1):
        cid = lax.axis_index("c")
        sid = lax.axis_index("s")
        wid = sid * _NC + cid
        base = wid * e_per_w
        s0 = sid * stripe
        zcp = pltpu.async_copy(z_hbm, acc.at[pl.ds(s0, stripe)], si0)
        slots = ((idx0, r0, si0, sr0), (idx1, r1, si1, sr1))
        npairs = (nch + 1) // 2

        def issue(k, slot):
            idx, rows, si, sr = slot
            off = base + k * c
            pltpu.async_copy(dst_hbm.at[pl.ds(off, c)], idx, si)
            pltpu.async_copy(ue_hbm.at[pl.ds(off, c)], rows, sr)

        def wait_loads(k, slot):
            idx, rows, si, sr = slot
            off = base + k * c
            pltpu.make_async_copy(dst_hbm.at[pl.ds(off, c)], idx, si).wait()
            pltpu.make_async_copy(ue_hbm.at[pl.ds(off, c)], rows, sr).wait()

        zcp.wait()
        plsc.subcore_barrier()
        issue(0, slots[0])
        issue(1, slots[1])

        @pl.loop(0, npairs)
        def _pair(jp):
            j0 = 2 * jp
            for si_ in range(2):
                j = j0 + si_
                slot = slots[si_]

                @pl.when(j < nch)
                def _():
                    wait_loads(j, slot)
                    pltpu.sync_copy(slot[1], acc.at[slot[0]], add=True)

                    @pl.when(j + 2 < nch)
                    def _():
                        issue(j + 2, slot)

        plsc.subcore_barrier()
        pltpu.sync_copy(acc.at[pl.ds(s0, stripe)],
                        out_hbm.at[cid, pl.ds(s0, stripe)])

    parts = sk(ue, dst, jnp.zeros((stripe, LATENT), jnp.float32))
    return parts[0], parts[1]


def kernel(x, edge_index, edge_attr, params):
    n = x.shape[0]
    src = edge_index[0].astype(jnp.int32)
    dst = edge_index[1].astype(jnp.int32)
    steps = len(params)
    a, b = _project(x, params[0]['edge']['W0'][:LATENT],
                    params[0]['edge']['W0'][LATENT:2 * LATENT])
    for s in range(steps):
        p = params[s]
        g = _gather_fused(a, b, src, dst)
        edge_attr = _edge_mlp(g, edge_attr, p['edge'])
        parts = _scatter_sum(edge_attr, dst, n)
        wnext = params[s + 1]['edge']['W0'] if s + 1 < steps else None
        x, a, b = _node_mlp(x, parts, p['node'], wnext)
    return (x, edge_attr)
